# baseline (speedup 1.0000x reference)
import jax
import jax.numpy as jnp
from jax.experimental import pallas as pl
from jax.experimental.pallas import tpu as pltpu

B, T, N, H, D, K, OW = 8, 12, 128, 128, 12, 3, 12
BN = B * N
O2 = 2 * H
O3 = 3 * H


def _mmb(a, w):
    return jnp.dot(a.reshape(BN, a.shape[-1]).astype(jnp.bfloat16), w,
                   preferred_element_type=jnp.float32)


def _fused_body(x_ref, sup_ref, emb_ref, fw0_ref,
                gwh0_ref, uwh0_ref, wg1_ref, wc1_ref, bias_ref,
                ew_ref, eb_ref, out_ref, h1_scr, h2_scr):
    t = pl.program_id(0)
    nt = pl.num_programs(0)

    @pl.when(t == 0)
    def _init():
        h1_scr[...] = jnp.zeros_like(h1_scr)
        h2_scr[...] = jnp.zeros_like(h2_scr)

    sup = [jnp.broadcast_to(sup_ref[k][None], (B, N, N)) for k in range(K)]
    emb = emb_ref[...]
    bias = bias_ref[...][None]

    def agg_cat(v3):
        ys = [
            jnp.einsum("bnm,bmf->bnf", sup[k], v3,
                       preferred_element_type=jnp.float32)
            for k in range(K)
        ]
        return jnp.concatenate(ys, axis=-1)

    def fold(t2, o):
        t3 = t2.reshape(B, N, D * o)
        acc = t3[..., :o] * emb[None, :, 0:1]
        for d in range(1, D):
            acc = acc + t3[..., d * o:(d + 1) * o] * emb[None, :, d:d + 1]
        return acc

    x3 = x_ref[...][..., None]
    yx0 = agg_cat(x3)
    xf = (yx0[..., 0:1] * fw0_ref[0][None]
          + yx0[..., 1:2] * fw0_ref[1][None]
          + yx0[..., 2:3] * fw0_ref[2][None])

    s1 = h1_scr[...].reshape(B, N, H)
    yx1 = agg_cat(s1)

    s2 = h2_scr[...].reshape(B, N, H)
    tg1 = _mmb(jnp.concatenate([yx1, agg_cat(s2)], axis=-1), wg1_ref[...])
    zr1 = jax.nn.sigmoid(fold(tg1, O2) + bias[..., O3:O3 + O2])
    z1 = zr1[..., :H]
    r1 = zr1[..., H:]
    tc1 = _mmb(jnp.concatenate([yx1, agg_cat(z1 * s2)], axis=-1), wc1_ref[...])
    hc1 = jnp.tanh(fold(tc1, H) + bias[..., O3 + O2:])
    h2 = r1 * s2 + (1.0 - r1) * hc1
    h2_scr[...] = jnp.where(t == 0, 0.0, h2.reshape(BN, H))

    tg = _mmb(yx1, gwh0_ref[...])
    zr = jax.nn.sigmoid(fold(tg, O2) + bias[..., :O2] + xf[..., :O2])
    z = zr[..., :H]
    r = zr[..., H:]
    tc = _mmb(agg_cat(z * s1), uwh0_ref[...])
    hc = jnp.tanh(fold(tc, H) + bias[..., O2:O3] + xf[..., O2:])
    h1 = r * s1 + (1.0 - r) * hc
    h1_scr[...] = h1.reshape(BN, H)

    @pl.when(t == nt - 1)
    def _head():
        o3 = jnp.dot(h2.reshape(BN, H), ew_ref[...],
                     preferred_element_type=jnp.float32) + eb_ref[...]
        out_ref[...] = jnp.transpose(
            o3.reshape(B, N, OW), (0, 2, 1))


def _pool2d(pool, lo, hi, o):
    return jnp.transpose(pool[:, :, lo:hi, :], (1, 2, 0, 3)).reshape(
        K * (hi - lo), D * o)


def kernel(batch_x, lap, l0_gate_w, l0_gate_b, l0_upd_w, l0_upd_b,
           l1_gate_w, l1_gate_b, l1_upd_w, l1_upd_b,
           node_emb, end_w, end_b):
    eye = jnp.eye(N, dtype=jnp.float32)
    apt = jax.nn.softmax(jax.nn.relu(node_emb @ node_emb.T), axis=1)
    supports = jnp.stack([eye, lap, apt], axis=0)

    wx0 = jnp.concatenate(
        [l0_gate_w[:, :, 0, :], l0_upd_w[:, :, 0, :]], axis=-1)
    fw0 = jnp.einsum("nd,dko->kno", node_emb, wx0)
    gwh0 = _pool2d(l0_gate_w, 1, 1 + H, O2).astype(jnp.bfloat16)
    uwh0 = _pool2d(l0_upd_w, 1, 1 + H, H).astype(jnp.bfloat16)

    wg1 = jnp.concatenate([_pool2d(l1_gate_w, 0, H, O2),
                           _pool2d(l1_gate_w, H, 2 * H, O2)],
                          axis=0).astype(jnp.bfloat16)
    wc1 = jnp.concatenate([_pool2d(l1_upd_w, 0, H, H),
                           _pool2d(l1_upd_w, H, 2 * H, H)],
                          axis=0).astype(jnp.bfloat16)

    bias = node_emb @ jnp.concatenate(
        [l0_gate_b, l0_upd_b, l1_gate_b, l1_upd_b], axis=1)

    ew = jnp.transpose(end_w)
    eb = end_b.reshape(1, OW)

    full2 = lambda t: (0, 0)
    full3 = lambda t: (0, 0, 0)

    out = pl.pallas_call(
        _fused_body,
        grid=(T + 1,),
        in_specs=[
            pl.BlockSpec((B, N), lambda t: (0, jnp.minimum(t, T - 1))),
            pl.BlockSpec(supports.shape, full3),
            pl.BlockSpec(node_emb.shape, full2),
            pl.BlockSpec(fw0.shape, full3),
            pl.BlockSpec(gwh0.shape, full2),
            pl.BlockSpec(uwh0.shape, full2),
            pl.BlockSpec(wg1.shape, full2),
            pl.BlockSpec(wc1.shape, full2),
            pl.BlockSpec(bias.shape, full2),
            pl.BlockSpec(ew.shape, full2),
            pl.BlockSpec(eb.shape, full2),
        ],
        out_specs=pl.BlockSpec((B, OW, N), full3),
        out_shape=jax.ShapeDtypeStruct((B, OW, N), jnp.float32),
        scratch_shapes=[pltpu.VMEM((BN, H), jnp.float32),
                        pltpu.VMEM((BN, H), jnp.float32)],
        compiler_params=pltpu.CompilerParams(
            dimension_semantics=("arbitrary",)),
    )(batch_x.reshape(B, T * N), supports, node_emb, fw0,
      gwh0, uwh0, wg1, wc1, bias, ew, eb)

    return out


# speedup vs baseline: 1.0767x; 1.0767x over previous
import jax
import jax.numpy as jnp
from jax.experimental import pallas as pl
from jax.experimental.pallas import tpu as pltpu

B, T, N, H, D, K, OW = 8, 12, 128, 128, 12, 3, 12
BN = B * N
O2 = 2 * H
O3 = 3 * H


def _mmb(a, w):
    return jnp.dot(a.reshape(BN, a.shape[-1]).astype(jnp.bfloat16), w,
                   preferred_element_type=jnp.float32)


def _fused_body(x_ref, lap_ref, emb_ref, fw0_ref,
                gwh0_ref, uwh0_ref, wg1_ref, wc1_ref, bias_ref,
                ew_ref, eb_ref, out_ref, h1_scr, h2_scr, apt_scr):
    t = pl.program_id(0)
    nt = pl.num_programs(0)
    emb = emb_ref[...]

    @pl.when(t == 0)
    def _init():
        h1_scr[...] = jnp.zeros_like(h1_scr)
        h2_scr[...] = jnp.zeros_like(h2_scr)
        sim = jax.nn.relu(jax.lax.dot_general(
            emb, emb, (((1,), (1,)), ((), ())),
            preferred_element_type=jnp.float32))
        e = jnp.exp(sim - jnp.max(sim, axis=1, keepdims=True))
        apt_scr[...] = e / jnp.sum(e, axis=1, keepdims=True)

    bias = bias_ref[...][None]
    supl = jnp.broadcast_to(lap_ref[...][None], (B, N, N))
    supa = jnp.broadcast_to(apt_scr[...][None], (B, N, N))

    def agg_cat(v3):
        return jnp.concatenate([
            v3,
            jnp.einsum("bnm,bmf->bnf", supl, v3,
                       preferred_element_type=jnp.float32),
            jnp.einsum("bnm,bmf->bnf", supa, v3,
                       preferred_element_type=jnp.float32),
        ], axis=-1)

    def fold(t2, o):
        t3 = t2.reshape(B, N, D * o)
        acc = t3[..., :o] * emb[None, :, 0:1]
        for d in range(1, D):
            acc = acc + t3[..., d * o:(d + 1) * o] * emb[None, :, d:d + 1]
        return acc

    x3 = x_ref[...][..., None]
    yx0 = agg_cat(x3)
    xf = (yx0[..., 0:1] * fw0_ref[0][None]
          + yx0[..., 1:2] * fw0_ref[1][None]
          + yx0[..., 2:3] * fw0_ref[2][None])

    s1 = h1_scr[...].reshape(B, N, H)
    tg = _mmb(agg_cat(s1), gwh0_ref[...])
    zr = jax.nn.sigmoid(fold(tg, O2) + bias[..., :O2] + xf[..., :O2])
    z = zr[..., :H]
    r = zr[..., H:]
    tc = _mmb(agg_cat(z * s1), uwh0_ref[...])
    hc = jnp.tanh(fold(tc, H) + bias[..., O2:O3] + xf[..., O2:])
    h1 = r * s1 + (1.0 - r) * hc
    h1_scr[...] = h1.reshape(BN, H)

    yx1 = agg_cat(h1)
    s2 = h2_scr[...].reshape(B, N, H)
    tg1 = _mmb(jnp.concatenate([yx1, agg_cat(s2)], axis=-1), wg1_ref[...])
    zr1 = jax.nn.sigmoid(fold(tg1, O2) + bias[..., O3:O3 + O2])
    z1 = zr1[..., :H]
    r1 = zr1[..., H:]
    tc1 = _mmb(jnp.concatenate([yx1, agg_cat(z1 * s2)], axis=-1), wc1_ref[...])
    hc1 = jnp.tanh(fold(tc1, H) + bias[..., O3 + O2:])
    h2 = r1 * s2 + (1.0 - r1) * hc1
    h2_scr[...] = h2.reshape(BN, H)

    @pl.when(t == nt - 1)
    def _head():
        o3 = jnp.dot(h2.reshape(BN, H), ew_ref[...],
                     preferred_element_type=jnp.float32) + eb_ref[...]
        out_ref[...] = jnp.transpose(
            o3.reshape(B, N, OW), (0, 2, 1))


def _pool2d(pool, lo, hi, o):
    return jnp.transpose(pool[:, :, lo:hi, :], (1, 2, 0, 3)).reshape(
        K * (hi - lo), D * o)


def kernel(batch_x, lap, l0_gate_w, l0_gate_b, l0_upd_w, l0_upd_b,
           l1_gate_w, l1_gate_b, l1_upd_w, l1_upd_b,
           node_emb, end_w, end_b):
    wx0 = jnp.concatenate(
        [l0_gate_w[:, :, 0, :], l0_upd_w[:, :, 0, :]], axis=-1)
    fw0 = jnp.einsum("nd,dko->kno", node_emb, wx0)
    gwh0 = _pool2d(l0_gate_w, 1, 1 + H, O2).astype(jnp.bfloat16)
    uwh0 = _pool2d(l0_upd_w, 1, 1 + H, H).astype(jnp.bfloat16)

    wg1 = jnp.concatenate([_pool2d(l1_gate_w, 0, H, O2),
                           _pool2d(l1_gate_w, H, 2 * H, O2)],
                          axis=0).astype(jnp.bfloat16)
    wc1 = jnp.concatenate([_pool2d(l1_upd_w, 0, H, H),
                           _pool2d(l1_upd_w, H, 2 * H, H)],
                          axis=0).astype(jnp.bfloat16)

    bias = node_emb @ jnp.concatenate(
        [l0_gate_b, l0_upd_b, l1_gate_b, l1_upd_b], axis=1)

    ew = jnp.transpose(end_w)
    eb = end_b.reshape(1, OW)

    full2 = lambda t: (0, 0)
    full3 = lambda t: (0, 0, 0)

    out = pl.pallas_call(
        _fused_body,
        grid=(T,),
        in_specs=[
            pl.BlockSpec((B, N), lambda t: (0, t)),
            pl.BlockSpec(lap.shape, full2),
            pl.BlockSpec(node_emb.shape, full2),
            pl.BlockSpec(fw0.shape, full3),
            pl.BlockSpec(gwh0.shape, full2),
            pl.BlockSpec(uwh0.shape, full2),
            pl.BlockSpec(wg1.shape, full2),
            pl.BlockSpec(wc1.shape, full2),
            pl.BlockSpec(bias.shape, full2),
            pl.BlockSpec(ew.shape, full2),
            pl.BlockSpec(eb.shape, full2),
        ],
        out_specs=pl.BlockSpec((B, OW, N), full3),
        out_shape=jax.ShapeDtypeStruct((B, OW, N), jnp.float32),
        scratch_shapes=[pltpu.VMEM((BN, H), jnp.float32),
                        pltpu.VMEM((BN, H), jnp.float32),
                        pltpu.VMEM((N, N), jnp.float32)],
        compiler_params=pltpu.CompilerParams(
            dimension_semantics=("arbitrary",)),
    )(batch_x.reshape(B, T * N), lap, node_emb, fw0,
      gwh0, uwh0, wg1, wc1, bias, ew, eb)

    return out


# speedup vs baseline: 1.0806x; 1.0037x over previous
import jax
import jax.numpy as jnp
from jax.experimental import pallas as pl
from jax.experimental.pallas import tpu as pltpu

B, T, N, H, D, K, OW = 8, 12, 128, 128, 12, 3, 12
BN = B * N
O2 = 2 * H
O3 = 3 * H


def _mmb(a, w):
    return jnp.dot(a.reshape(BN, a.shape[-1]).astype(jnp.bfloat16), w,
                   preferred_element_type=jnp.float32)


def _fused_body(x_ref, lap_ref, emb_ref, fw0_ref,
                gwh0_ref, uwh0_ref, wg1_ref, wc1_ref, bias_ref,
                ew_ref, eb_ref, out_ref, h1_scr, h2_scr, apt_scr):
    t = pl.program_id(0)
    nt = pl.num_programs(0)
    emb = emb_ref[...]

    @pl.when(t == 0)
    def _init():
        h1_scr[...] = jnp.zeros_like(h1_scr)
        h2_scr[...] = jnp.zeros_like(h2_scr)
        sim = jax.nn.relu(jax.lax.dot_general(
            emb, emb, (((1,), (1,)), ((), ())),
            preferred_element_type=jnp.float32))
        e = jnp.exp(sim - jnp.max(sim, axis=1, keepdims=True))
        apt_scr[...] = (e / jnp.sum(e, axis=1, keepdims=True)).astype(
            jnp.bfloat16)

    bias = bias_ref[...][None]
    supl = jnp.broadcast_to(lap_ref[...][None], (B, N, N))
    supa = jnp.broadcast_to(apt_scr[...][None], (B, N, N))

    def agg_cat(v3):
        v3b = v3.astype(jnp.bfloat16)
        return jnp.concatenate([
            v3,
            jnp.einsum("bnm,bmf->bnf", supl, v3b,
                       preferred_element_type=jnp.float32),
            jnp.einsum("bnm,bmf->bnf", supa, v3b,
                       preferred_element_type=jnp.float32),
        ], axis=-1)

    def fold(t2, o):
        t3 = t2.reshape(B, N, D * o)
        acc = t3[..., :o] * emb[None, :, 0:1]
        for d in range(1, D):
            acc = acc + t3[..., d * o:(d + 1) * o] * emb[None, :, d:d + 1]
        return acc

    x3 = x_ref[...][..., None]
    yx0 = agg_cat(x3)
    xf = (yx0[..., 0:1] * fw0_ref[0][None]
          + yx0[..., 1:2] * fw0_ref[1][None]
          + yx0[..., 2:3] * fw0_ref[2][None])

    s1 = h1_scr[...].reshape(B, N, H)
    tg = _mmb(agg_cat(s1), gwh0_ref[...])
    zr = jax.nn.sigmoid(fold(tg, O2) + bias[..., :O2] + xf[..., :O2])
    z = zr[..., :H]
    r = zr[..., H:]
    tc = _mmb(agg_cat(z * s1), uwh0_ref[...])
    hc = jnp.tanh(fold(tc, H) + bias[..., O2:O3] + xf[..., O2:])
    h1 = r * s1 + (1.0 - r) * hc
    h1_scr[...] = h1.reshape(BN, H)

    yx1 = agg_cat(h1)
    s2 = h2_scr[...].reshape(B, N, H)
    tg1 = _mmb(jnp.concatenate([yx1, agg_cat(s2)], axis=-1), wg1_ref[...])
    zr1 = jax.nn.sigmoid(fold(tg1, O2) + bias[..., O3:O3 + O2])
    z1 = zr1[..., :H]
    r1 = zr1[..., H:]
    tc1 = _mmb(jnp.concatenate([yx1, agg_cat(z1 * s2)], axis=-1), wc1_ref[...])
    hc1 = jnp.tanh(fold(tc1, H) + bias[..., O3 + O2:])
    h2 = r1 * s2 + (1.0 - r1) * hc1
    h2_scr[...] = h2.reshape(BN, H)

    @pl.when(t == nt - 1)
    def _head():
        o3 = jnp.dot(h2.reshape(BN, H), ew_ref[...],
                     preferred_element_type=jnp.float32) + eb_ref[...]
        out_ref[...] = jnp.transpose(
            o3.reshape(B, N, OW), (0, 2, 1))


def _pool2d(pool, lo, hi, o):
    return jnp.transpose(pool[:, :, lo:hi, :], (1, 2, 0, 3)).reshape(
        K * (hi - lo), D * o)


def kernel(batch_x, lap, l0_gate_w, l0_gate_b, l0_upd_w, l0_upd_b,
           l1_gate_w, l1_gate_b, l1_upd_w, l1_upd_b,
           node_emb, end_w, end_b):
    wx0 = jnp.concatenate(
        [l0_gate_w[:, :, 0, :], l0_upd_w[:, :, 0, :]], axis=-1)
    fw0 = jnp.einsum("nd,dko->kno", node_emb, wx0)
    gwh0 = _pool2d(l0_gate_w, 1, 1 + H, O2).astype(jnp.bfloat16)
    uwh0 = _pool2d(l0_upd_w, 1, 1 + H, H).astype(jnp.bfloat16)

    wg1 = jnp.concatenate([_pool2d(l1_gate_w, 0, H, O2),
                           _pool2d(l1_gate_w, H, 2 * H, O2)],
                          axis=0).astype(jnp.bfloat16)
    wc1 = jnp.concatenate([_pool2d(l1_upd_w, 0, H, H),
                           _pool2d(l1_upd_w, H, 2 * H, H)],
                          axis=0).astype(jnp.bfloat16)

    bias = node_emb @ jnp.concatenate(
        [l0_gate_b, l0_upd_b, l1_gate_b, l1_upd_b], axis=1)

    ew = jnp.transpose(end_w)
    eb = end_b.reshape(1, OW)
    lap = lap.astype(jnp.bfloat16)

    full2 = lambda t: (0, 0)
    full3 = lambda t: (0, 0, 0)

    out = pl.pallas_call(
        _fused_body,
        grid=(T,),
        in_specs=[
            pl.BlockSpec((B, N), lambda t: (0, t)),
            pl.BlockSpec(lap.shape, full2),
            pl.BlockSpec(node_emb.shape, full2),
            pl.BlockSpec(fw0.shape, full3),
            pl.BlockSpec(gwh0.shape, full2),
            pl.BlockSpec(uwh0.shape, full2),
            pl.BlockSpec(wg1.shape, full2),
            pl.BlockSpec(wc1.shape, full2),
            pl.BlockSpec(bias.shape, full2),
            pl.BlockSpec(ew.shape, full2),
            pl.BlockSpec(eb.shape, full2),
        ],
        out_specs=pl.BlockSpec((B, OW, N), full3),
        out_shape=jax.ShapeDtypeStruct((B, OW, N), jnp.float32),
        scratch_shapes=[pltpu.VMEM((BN, H), jnp.float32),
                        pltpu.VMEM((BN, H), jnp.float32),
                        pltpu.VMEM((N, N), jnp.bfloat16)],
        compiler_params=pltpu.CompilerParams(
            dimension_semantics=("arbitrary",)),
    )(batch_x.reshape(B, T * N), lap, node_emb, fw0,
      gwh0, uwh0, wg1, wc1, bias, ew, eb)

    return out
